# Initial kernel scaffold; baseline (speedup 1.0000x reference)
import jax
import jax.numpy as jnp
from jax.experimental import pallas as pl


def kernel(X, pos_enc, Wpos, bpos, Wq, bq, Wk, bk, Wv, bv, Wo, bo, Wpred, bpred, edge_index, graph_ids):
    raise NotImplementedError("write your pallas kernel here")


# speedup vs baseline: 3.6866x; 3.6866x over previous
import jax
import jax.numpy as jnp
from jax.experimental import pallas as pl

_N = 10000
_H = 256
_NH = 8
_HD = _H // _NH
_L = 8
_G = 64


def _mm_kernel(x_ref, w_ref, b_ref, o_ref):
    o_ref[...] = (
        jnp.dot(x_ref[...], w_ref[...], preferred_element_type=jnp.float32)
        + b_ref[...]
    )


def _mm(x, w, b, bn):
    n, kin = x.shape
    kout = w.shape[1]
    return pl.pallas_call(
        _mm_kernel,
        grid=(n // bn,),
        in_specs=[
            pl.BlockSpec((bn, kin), lambda i: (i, 0)),
            pl.BlockSpec((kin, kout), lambda i: (0, 0)),
            pl.BlockSpec((1, kout), lambda i: (0, 0)),
        ],
        out_specs=pl.BlockSpec((bn, kout), lambda i: (i, 0)),
        out_shape=jax.ShapeDtypeStruct((n, kout), jnp.float32),
    )(x, w, b.reshape(1, kout))


def _sddmm_kernel(qr_ref, kc_ref, o_ref):
    prod = qr_ref[...] * kc_ref[...]
    be = prod.shape[0]
    o_ref[...] = prod.reshape(be, _HD, _NH).sum(axis=1)


def _sddmm(qr, kc, be):
    e = qr.shape[0]
    return pl.pallas_call(
        _sddmm_kernel,
        grid=(e // be,),
        in_specs=[
            pl.BlockSpec((be, _H), lambda i: (i, 0)),
            pl.BlockSpec((be, _H), lambda i: (i, 0)),
        ],
        out_specs=pl.BlockSpec((be, _NH), lambda i: (i, 0)),
        out_shape=jax.ShapeDtypeStruct((e, _NH), jnp.float32),
    )(qr, kc)


def _av_kernel(attn_ref, vc_ref, o_ref):
    be = vc_ref.shape[0]
    a = attn_ref[...]
    v = vc_ref[...].reshape(be, _HD, _NH)
    o_ref[...] = (a[:, None, :] * v).reshape(be, _H)


def _av(attn, vc, be):
    e = attn.shape[0]
    return pl.pallas_call(
        _av_kernel,
        grid=(e // be,),
        in_specs=[
            pl.BlockSpec((be, _NH), lambda i: (i, 0)),
            pl.BlockSpec((be, _H), lambda i: (i, 0)),
        ],
        out_specs=pl.BlockSpec((be, _H), lambda i: (i, 0)),
        out_shape=jax.ShapeDtypeStruct((e, _H), jnp.float32),
    )(attn, vc)


def _pool_kernel(h_ref, gid_ref, wp_ref, bp_ref, o_ref):
    gid = gid_ref[...]
    npad = gid.shape[1]
    iota = jax.lax.broadcasted_iota(jnp.int32, (_G, npad), 0)
    onehot = (iota == gid).astype(jnp.float32)
    pooled = jnp.dot(onehot, h_ref[...], preferred_element_type=jnp.float32)
    o_ref[...] = (
        jnp.dot(pooled, wp_ref[...], preferred_element_type=jnp.float32)
        + bp_ref[...]
    )


def kernel(X, pos_enc, Wpos, bpos, Wq, bq, Wk, bk, Wv, bv, Wo, bo, Wpred, bpred, edge_index, graph_ids):
    n = pos_enc.shape[0]
    e = edge_index.shape[1]
    row = edge_index[0]
    col = edge_index[1]
    scaling = float(_HD) ** -0.5

    h = _mm(pos_enc, Wpos, bpos, bn=1000)

    for l in range(_L):
        wqkv = jnp.concatenate([Wq[l], Wk[l], Wv[l]], axis=1)
        bqkv = jnp.concatenate([bq[l], bk[l], bv[l]], axis=0)
        qkv = _mm(h, wqkv, bqkv, bn=1000)
        q = qkv[:, :_H] * scaling
        k = qkv[:, _H:2 * _H]
        v = qkv[:, 2 * _H:]

        scores = _sddmm(q[row], k[col], be=2000)
        m = jax.ops.segment_max(scores, row, num_segments=n)
        m = jnp.where(jnp.isfinite(m), m, 0.0)
        ex = jnp.exp(scores - m[row])
        denom = jax.ops.segment_sum(ex, row, num_segments=n)
        attn = ex / (denom[row] + 1e-9)
        contrib = _av(attn, v[col], be=2000)
        out = jax.ops.segment_sum(contrib, row, num_segments=n)
        h = _mm(out, Wo[l], bo[l], bn=1000)

    npad = 10240
    h_pad = jnp.zeros((npad, _H), jnp.float32).at[:n].set(h)
    gid_pad = jnp.full((1, npad), _G, jnp.int32).at[0, :n].set(graph_ids)
    out_dim = Wpred.shape[1]
    res = pl.pallas_call(
        _pool_kernel,
        in_specs=[
            pl.BlockSpec((npad, _H), lambda: (0, 0)),
            pl.BlockSpec((1, npad), lambda: (0, 0)),
            pl.BlockSpec((_H, out_dim), lambda: (0, 0)),
            pl.BlockSpec((1, out_dim), lambda: (0, 0)),
        ],
        out_specs=pl.BlockSpec((_G, out_dim), lambda: (0, 0)),
        out_shape=jax.ShapeDtypeStruct((_G, out_dim), jnp.float32),
    )(h_pad, gid_pad, Wpred, bpred.reshape(1, out_dim))
    return res
